# baseline (speedup 1.0000x reference)
import jax
import jax.numpy as jnp
import numpy as np
from jax.experimental import pallas as pl
from jax.experimental.pallas import tpu as pltpu

_N = 100000
_DIM = 128
_ATOM_DIM = 6
_MAX_SPD = 16
_MAX_DIS = 20
_KGROUPS = 24
_JK = _KGROUPS * _MAX_SPD
_K1 = 64 + _ATOM_DIM + 2
_ZW = _DIM + _JK
_BLOCK = 10000

_L = np.array(list(range(_MAX_DIS + 1)) + [1001, 1002, 2**24], np.float64)
_T1 = np.where(_L <= 1001, 1.0 - _L, np.where(_L == 1002, -1000.0, -(2.0**24)))
_T2 = np.where(_L == 1002, -1.0, 0.0)

_M1_STATIC = np.zeros((_K1, _ZW), np.float32)
for _j in range(_MAX_SPD):
    for _k in range(_KGROUPS):
        _c = _DIM + _k * _MAX_SPD + _j
        _M1_STATIC[_j, _c] = 1.0
        _M1_STATIC[32 + _j, _c] = 256.0
for _k in range(_KGROUPS):
    _M1_STATIC[_K1 - 2, _DIM + _k * _MAX_SPD : _DIM + (_k + 1) * _MAX_SPD] = _T1[_k]
    _M1_STATIC[_K1 - 1, _DIM + _k * _MAX_SPD : _DIM + (_k + 1) * _MAX_SPD] = _T2[_k]


def _body(d_ref, xo_ref, m1_ref, m2_ref, b2_ref, o_ref):
    d = d_ref[...]
    d_lo = (d & 255).astype(jnp.bfloat16)
    d_hi = (d >> 8).astype(jnp.bfloat16)
    a1 = jnp.concatenate([d_lo, d_hi, xo_ref[...]], axis=1)

    z = jnp.dot(a1, m1_ref[...], preferred_element_type=jnp.float32)
    lane = jax.lax.broadcasted_iota(jnp.int32, (1, _ZW), 1)
    cap = jnp.where(lane < _DIM, jnp.inf, 1.0).astype(jnp.bfloat16)
    g = jnp.minimum(jnp.maximum(z.astype(jnp.bfloat16), jnp.bfloat16(0.0)), cap)
    o_ref[...] = (
        jnp.dot(g, m2_ref[...], preferred_element_type=jnp.float32)
        + b2_ref[...][None, :]
    )


@jax.jit
def kernel(x, d, W1, b1, W2, b2, table):
    ones2 = jnp.ones((_N, 2), jnp.float32)
    xo = jnp.concatenate([x, ones2], axis=1).astype(jnp.bfloat16)

    m1 = jnp.asarray(_M1_STATIC)
    m1 = m1.at[64 : 64 + _ATOM_DIM, :_DIM].set(W1)
    m1 = m1.at[_K1 - 2, :_DIM].set(b1)
    m1 = m1.astype(jnp.bfloat16)

    td = table - jnp.concatenate([jnp.zeros((1, _DIM), table.dtype), table[:-1]], axis=0)
    td = jnp.concatenate([td, jnp.zeros((1, _DIM), table.dtype)], axis=0)
    td_rep = jnp.repeat(td, _MAX_SPD, axis=0)
    m2 = jnp.concatenate([W2, td_rep], axis=0).astype(jnp.bfloat16)

    grid = (_N // _BLOCK,)
    return pl.pallas_call(
        _body,
        grid=grid,
        in_specs=[
            pl.BlockSpec((_BLOCK, 32), lambda i: (i, 0)),
            pl.BlockSpec((_BLOCK, _ATOM_DIM + 2), lambda i: (i, 0)),
            pl.BlockSpec((_K1, _ZW), lambda i: (0, 0)),
            pl.BlockSpec((_ZW, _DIM), lambda i: (0, 0)),
            pl.BlockSpec((_DIM,), lambda i: (0,)),
        ],
        out_specs=pl.BlockSpec((_BLOCK, _DIM), lambda i: (i, 0)),
        out_shape=jax.ShapeDtypeStruct((_N, _DIM), jnp.float32),
        compiler_params=pltpu.CompilerParams(
            dimension_semantics=("parallel",),
        ),
    )(d, xo, m1, m2, b2)


# speedup vs baseline: 1.3394x; 1.3394x over previous
import jax
import jax.numpy as jnp
import numpy as np
from jax.experimental import pallas as pl
from jax.experimental.pallas import tpu as pltpu

_N = 100000
_DIM = 128
_ATOM_DIM = 6
_MAX_SPD = 16
_MAX_DIS = 20
_KGROUPS = 24
_JK = _KGROUPS * _MAX_SPD
_K1 = _ATOM_DIM + 2 * _MAX_SPD + 2
_ZW = _DIM + _JK
_BLOCK = 10000

_L = np.array(list(range(_MAX_DIS + 1)) + [1001, 1002, 2**24], np.float64)
_T1 = np.where(_L <= 1001, 1.0 - _L, np.where(_L == 1002, -1000.0, -(2.0**24)))
_T2 = np.where(_L == 1002, -1.0, 0.0)

_M1_STATIC = np.zeros((_K1, _ZW), np.float32)
for _j in range(_MAX_SPD):
    for _k in range(_KGROUPS):
        _c = _DIM + _k * _MAX_SPD + _j
        _M1_STATIC[_ATOM_DIM + _j, _c] = 1.0
        _M1_STATIC[_ATOM_DIM + _MAX_SPD + _j, _c] = 256.0
for _k in range(_KGROUPS):
    _M1_STATIC[_K1 - 2, _DIM + _k * _MAX_SPD : _DIM + (_k + 1) * _MAX_SPD] = _T1[_k]
    _M1_STATIC[_K1 - 1, _DIM + _k * _MAX_SPD : _DIM + (_k + 1) * _MAX_SPD] = _T2[_k]

_CAP = np.where(np.arange(_ZW) < _DIM, np.inf, 1.0).astype(np.float32)


def _body(a_ref, m1_ref, m2_ref, b2_ref, o_ref):
    z = jnp.dot(a_ref[...], m1_ref[...], preferred_element_type=jnp.float32)
    lane = jax.lax.broadcasted_iota(jnp.int32, (1, _ZW), 1)
    cap = jnp.where(lane < _DIM, jnp.inf, 1.0).astype(jnp.bfloat16)
    g = jnp.minimum(jnp.maximum(z.astype(jnp.bfloat16), jnp.bfloat16(0.0)), cap)
    o_ref[...] = (
        jnp.dot(g, m2_ref[...], preferred_element_type=jnp.float32)
        + b2_ref[...][None, :]
    )


@jax.jit
def kernel(x, d, W1, b1, W2, b2, table):
    d16 = d[:, :_MAX_SPD]
    d_hi = d16 >> 8
    d_lo = d16 & 255
    ones2 = jnp.ones((_N, 2), jnp.bfloat16)
    a1 = jnp.concatenate(
        [
            x.astype(jnp.bfloat16),
            d_lo.astype(jnp.bfloat16),
            d_hi.astype(jnp.bfloat16),
            ones2,
        ],
        axis=1,
    )

    m1 = jnp.asarray(_M1_STATIC)
    m1 = m1.at[:_ATOM_DIM, :_DIM].set(W1)
    m1 = m1.at[_K1 - 2, :_DIM].set(b1)
    m1 = m1.astype(jnp.bfloat16)

    td = table - jnp.concatenate([jnp.zeros((1, _DIM), table.dtype), table[:-1]], axis=0)
    td = jnp.concatenate([td, jnp.zeros((1, _DIM), table.dtype)], axis=0)
    td_rep = jnp.repeat(td, _MAX_SPD, axis=0)
    m2 = jnp.concatenate([W2, td_rep], axis=0).astype(jnp.bfloat16)

    grid = (_N // _BLOCK,)
    return pl.pallas_call(
        _body,
        grid=grid,
        in_specs=[
            pl.BlockSpec((_BLOCK, _K1), lambda i: (i, 0)),
            pl.BlockSpec((_K1, _ZW), lambda i: (0, 0)),
            pl.BlockSpec((_ZW, _DIM), lambda i: (0, 0)),
            pl.BlockSpec((_DIM,), lambda i: (0,)),
        ],
        out_specs=pl.BlockSpec((_BLOCK, _DIM), lambda i: (i, 0)),
        out_shape=jax.ShapeDtypeStruct((_N, _DIM), jnp.float32),
        compiler_params=pltpu.CompilerParams(
            dimension_semantics=("parallel",),
        ),
    )(a1, m1, m2, b2)
